# Initial kernel scaffold; baseline (speedup 1.0000x reference)
import jax
import jax.numpy as jnp
from jax.experimental import pallas as pl


def kernel(edge_index, num_nodes, enc1, enc2):
    raise NotImplementedError("write your pallas kernel here")


# speedup vs baseline: 1.9326x; 1.9326x over previous
import functools

import jax
import jax.numpy as jnp
from jax import lax
from jax.experimental import pallas as pl
from jax.experimental.pallas import tpu as pltpu
from jax.experimental.pallas import tpu_sc as plsc

MAXD = 512
D = 128
N_NODES = 50000
N_EDGES = 1600000

NC = 2
NS = 16
NW = NC * NS
L = 16

E_PAD = 1638400
N_CHUNKS = E_PAD // 128
CHUNKS_PER_TILE = N_CHUNKS // NS
BIG = 5
BIG_CHUNKS = CHUNKS_PER_TILE // BIG
PADV = N_NODES

HIST = 50176
SLICE = HIST // NS
NODES_PER_W = HIST // NW
K = 112
SUBS = NODES_PER_W // K

_mesh = plsc.VectorSubcoreMesh(
    core_axis_name="c", subcore_axis_name="s", num_cores=NC, num_subcores=NS
)


@functools.partial(
    pl.kernel,
    out_type=jax.ShapeDtypeStruct((NC * HIST,), jnp.int32),
    mesh=_mesh,
    scratch_types=[
        pltpu.VMEM_SHARED((HIST,), jnp.int32),
        pltpu.VMEM((BIG_CHUNKS, 128), jnp.int32),
        pltpu.VMEM((128,), jnp.int32),
        pltpu.VMEM((SLICE,), jnp.int32),
        pltpu.VMEM((L,), jnp.int32),
    ],
)
def _hist_kernel(ei_hbm, nn_hbm, hist_hbm, sp_hist, stage, ones, cbuf, nn_v):
    c = lax.axis_index("c")
    s = lax.axis_index("s")

    def _zero(i, _):
        cbuf[pl.ds(i * L, L)] = jnp.zeros((L,), jnp.int32)
        return 0

    lax.fori_loop(0, SLICE // L, _zero, 0)
    pltpu.sync_copy(cbuf, sp_hist.at[pl.ds(s * SLICE, SLICE)])

    for i in range(128 // L):
        ones[pl.ds(i * L, L)] = jnp.ones((L,), jnp.int32)
    pltpu.sync_copy(nn_hbm, nn_v)

    plsc.subcore_barrier()

    base = s * CHUNKS_PER_TILE
    for big in range(BIG):
        pltpu.sync_copy(
            ei_hbm.at[c, pl.ds(base + big * BIG_CHUNKS, BIG_CHUNKS)], stage
        )

        def _scat(j, _):
            pltpu.sync_copy(ones, sp_hist.at[stage.at[j]], add=True)
            return 0

        lax.fori_loop(0, BIG_CHUNKS, _scat, 0)

    plsc.subcore_barrier()

    pltpu.sync_copy(sp_hist.at[pl.ds(s * SLICE, SLICE)], cbuf)
    nn = nn_v[...]

    def _clip(k, _):
        v = cbuf[pl.ds(k * L, L)]
        pos = s * SLICE + k * L + lax.iota(jnp.int32, L)
        cbuf[pl.ds(k * L, L)] = jnp.where(pos < nn, jnp.minimum(v, MAXD), 0)
        return 0

    lax.fori_loop(0, SLICE // L, _clip, 0)
    pltpu.sync_copy(cbuf, hist_hbm.at[pl.ds(c * HIST + s * SLICE, SLICE)])


@functools.partial(
    pl.kernel,
    out_type=jax.ShapeDtypeStruct((HIST, D), jnp.float32),
    mesh=_mesh,
    scratch_types=[
        pltpu.VMEM_SHARED((MAXD + 1, D), jnp.float32),
        pltpu.VMEM_SHARED((MAXD + 1, D), jnp.float32),
        pltpu.VMEM((NODES_PER_W,), jnp.int32),
        pltpu.VMEM((NODES_PER_W,), jnp.int32),
        pltpu.VMEM((K, D), jnp.float32),
        pltpu.VMEM((D,), jnp.float32),
    ],
)
def _embed_kernel(hist_hbm, e1_hbm, e2_hbm, out_hbm, t1, t2, idx1, idx2, acc, zrow):
    c = lax.axis_index("c")
    s = lax.axis_index("s")
    w = s * NC + c

    @pl.when(s == 0)
    def _stage_tables():
        pltpu.sync_copy(e1_hbm, t1)
        pltpu.sync_copy(e2_hbm, t2)
        for i in range(D // L):
            zrow[pl.ds(i * L, L)] = jnp.zeros((L,), jnp.float32)
        pltpu.sync_copy(zrow, t1.at[0])
        pltpu.sync_copy(zrow, t2.at[0])

    plsc.subcore_barrier()

    pltpu.sync_copy(hist_hbm.at[pl.ds(w * NODES_PER_W, NODES_PER_W)], idx1)
    pltpu.sync_copy(hist_hbm.at[pl.ds(HIST + w * NODES_PER_W, NODES_PER_W)], idx2)

    for sub in range(SUBS):
        pltpu.sync_copy(t1.at[idx1.at[pl.ds(sub * K, K)]], acc)
        pltpu.sync_copy(t2.at[idx2.at[pl.ds(sub * K, K)]], acc, add=True)
        pltpu.sync_copy(acc, out_hbm.at[pl.ds(w * NODES_PER_W + sub * K, K)])


def kernel(edge_index, num_nodes, enc1, enc2):
    ei = jnp.stack([edge_index[1], edge_index[0]])
    pad = jnp.full((2, E_PAD - N_EDGES), PADV, jnp.int32)
    ei = jnp.concatenate([ei, pad], axis=1).reshape(NC, N_CHUNKS, 128)
    nn = jnp.full((L,), num_nodes, jnp.int32)
    hist = _hist_kernel(ei, nn)
    emb = _embed_kernel(hist, enc1, enc2)
    return emb[:N_NODES]


# speedup vs baseline: 6.0082x; 3.1088x over previous
import functools

import jax
import jax.numpy as jnp
from jax import lax
from jax.experimental import pallas as pl
from jax.experimental.pallas import tpu as pltpu
from jax.experimental.pallas import tpu_sc as plsc

MAXD = 512
D = 128
N_NODES = 50000
N_EDGES = 1600000

NC = 2
NS = 16
NW = NC * NS
L = 16

EPT = N_EDGES // NS
SUB_E = 92 * 128
LAST_E = 46 * 128
ROUND_E = [SUB_E] * 8 + [LAST_E]
UNROLL = 23
LASTV = (EPT - sum(ROUND_E[:-1])) // L

HIST = 50176
SLICE = HIST // NS
GRP1 = 8

NPW = 1560
KK = 120
SUBS = NPW // KK
EXTRA = N_NODES - NW * NPW

_mesh = plsc.VectorSubcoreMesh(
    core_axis_name="c", subcore_axis_name="s", num_cores=NC, num_subcores=NS
)


@functools.partial(
    pl.kernel,
    out_type=jax.ShapeDtypeStruct((NC * HIST,), jnp.int32),
    mesh=_mesh,
    compiler_params=pltpu.CompilerParams(needs_layout_passes=False),
    scratch_types=[
        pltpu.VMEM((HIST,), jnp.int32),
        pltpu.VMEM((SUB_E,), jnp.int32),
        pltpu.VMEM((SUB_E,), jnp.int32),
        pltpu.VMEM((SLICE,), jnp.int32),
        [pltpu.VMEM((SLICE,), jnp.int32)] * GRP1,
        pltpu.VMEM((L,), jnp.int32),
        pltpu.VMEM_SHARED((GRP1 * HIST,), jnp.int32),
        pltpu.SemaphoreType.DMA,
        pltpu.SemaphoreType.DMA,
        [pltpu.SemaphoreType.DMA] * GRP1,
    ],
)
def _hist_kernel(
    ei_hbm, nn_hbm, hist_hbm, ph, st0, st1, cbuf, rbufs, nn_v, parts,
    sg0, sg1, rsems,
):
    c = lax.axis_index("c")
    s = lax.axis_index("s")

    row = 1 - c
    c0 = (s * EPT) // 128
    dv = (s * EPT - c0 * 128) // L
    base = c0 * 128
    stages, gsems = [st0, st1], [sg0, sg1]
    roff = [base + sum(ROUND_E[:r]) for r in range(len(ROUND_E))]
    gdescs = [
        pltpu.async_copy(
            ei_hbm.at[row, pl.ds(roff[0], ROUND_E[0])],
            st0.at[pl.ds(0, ROUND_E[0])],
            sg0,
        ),
        None,
    ]

    zero16 = jnp.zeros((L,), jnp.int32)

    def _zero(i, _):
        for u in range(8):
            ph[pl.ds((i * 8 + u) * L, L)] = zero16
        return 0

    lax.fori_loop(0, HIST // (L * 8), _zero, 0)
    pltpu.sync_copy(nn_hbm, nn_v)

    ones16 = jnp.ones((L,), jnp.int32)
    nrounds = len(ROUND_E)
    for rnd in range(nrounds):
        p = rnd & 1
        gdescs[p].wait()
        if rnd + 1 < nrounds:
            gdescs[1 - p] = pltpu.async_copy(
                ei_hbm.at[row, pl.ds(roff[rnd + 1], ROUND_E[rnd + 1])],
                stages[1 - p].at[pl.ds(0, ROUND_E[rnd + 1])],
                gsems[1 - p],
            )
        stg = stages[p]
        edge_rnd = rnd == 0 or rnd == nrounds - 1

        def _scat(i, _, stg=stg, rnd=rnd, edge_rnd=edge_rnd):
            vs = [stg[pl.ds((i * UNROLL + u) * L, L)] for u in range(UNROLL)]
            for u, v in enumerate(vs):
                if edge_rnd:
                    vpos = i * UNROLL + u
                    if rnd == 0:
                        ok = vpos >= dv
                    else:
                        ok = vpos < LASTV + dv
                    plsc.addupdate_scatter(
                        ph, [v], ones16, mask=jnp.full((L,), ok)
                    )
                else:
                    plsc.addupdate_scatter(ph, [v], ones16)
            return 0

        lax.fori_loop(0, ROUND_E[rnd] // (L * UNROLL), _scat, 0)

    for rnd, (lo, cnt) in enumerate([(0, GRP1), (GRP1, NS - GRP1)]):

        @pl.when((s >= lo) & (s < lo + cnt))
        def _publish():
            pltpu.sync_copy(ph, parts.at[pl.ds((s - lo) * HIST, HIST)])

        plsc.subcore_barrier()

        rdescs = [
            pltpu.async_copy(
                parts.at[pl.ds(t * HIST + s * SLICE, SLICE)], rbufs[t], rsems[t]
            )
            for t in range(cnt)
        ]
        for d in rdescs:
            d.wait()

        def _acc(k, _, rnd=rnd, cnt=cnt):
            for u in range(2):
                o = (k * 2 + u) * L
                v = rbufs[0][pl.ds(o, L)]
                for t in range(1, cnt):
                    v = v + rbufs[t][pl.ds(o, L)]
                if rnd == 1:
                    v = v + cbuf[pl.ds(o, L)]
                cbuf[pl.ds(o, L)] = v
            return 0

        lax.fori_loop(0, SLICE // (L * 2), _acc, 0)

        plsc.subcore_barrier()

    nn = nn_v[...]

    def _clip(k, _):
        for u in range(4):
            o = (k * 4 + u) * L
            v = cbuf[pl.ds(o, L)]
            pos = s * SLICE + o + lax.iota(jnp.int32, L)
            cbuf[pl.ds(o, L)] = jnp.where(pos < nn, jnp.minimum(v, MAXD), 0)
        return 0

    lax.fori_loop(0, SLICE // (L * 4), _clip, 0)
    pltpu.sync_copy(cbuf, hist_hbm.at[pl.ds(c * HIST + s * SLICE, SLICE)])


@functools.partial(
    pl.kernel,
    out_type=jax.ShapeDtypeStruct((N_NODES, D), jnp.float32),
    mesh=_mesh,
    scratch_types=[
        pltpu.VMEM_SHARED((MAXD + 1, D), jnp.float32),
        pltpu.VMEM_SHARED((MAXD + 1, D), jnp.float32),
        pltpu.VMEM((NPW + EXTRA,), jnp.int32),
        pltpu.VMEM((NPW + EXTRA,), jnp.int32),
        pltpu.VMEM((KK, D), jnp.float32),
        pltpu.VMEM((KK, D), jnp.float32),
        pltpu.VMEM((D,), jnp.float32),
        pltpu.SemaphoreType.DMA,
        pltpu.SemaphoreType.DMA,
        pltpu.SemaphoreType.DMA,
        pltpu.SemaphoreType.DMA,
        pltpu.SemaphoreType.DMA,
    ],
)
def _embed_kernel(
    hist_hbm, e1_hbm, e2_hbm, out_hbm, t1, t2, idx1, idx2, acc0, acc1, zrow,
    sem1, sem2, sem3, semi1, semi2,
):
    c = lax.axis_index("c")
    s = lax.axis_index("s")
    w = s * NC + c

    nload = NPW + EXTRA
    di1 = pltpu.async_copy(hist_hbm.at[pl.ds(w * NPW, nload)], idx1, semi1)
    di2 = pltpu.async_copy(
        hist_hbm.at[pl.ds(HIST + w * NPW, nload)], idx2, semi2
    )

    @pl.when(s == 0)
    def _stage_tables():
        pltpu.sync_copy(e1_hbm, t1)
        pltpu.sync_copy(e2_hbm, t2)
        for i in range(D // L):
            zrow[pl.ds(i * L, L)] = jnp.zeros((L,), jnp.float32)
        pltpu.sync_copy(zrow, t1.at[0])
        pltpu.sync_copy(zrow, t2.at[0])

    plsc.subcore_barrier()
    di1.wait()
    di2.wait()

    accs = [acc0, acc1]
    d1 = [
        pltpu.async_copy(t1.at[idx1.at[pl.ds(0, KK)]], acc0, sem1),
        None,
    ]
    dst = [None, None]
    for sub in range(SUBS):
        p = sub & 1
        d1[p].wait()
        d2 = pltpu.async_copy(
            t2.at[idx2.at[pl.ds(sub * KK, KK)]], accs[p], sem2, add=True
        )
        if sub + 1 < SUBS:
            if sub >= 1:
                dst[1 - p].wait()
            d1[1 - p] = pltpu.async_copy(
                t1.at[idx1.at[pl.ds((sub + 1) * KK, KK)]], accs[1 - p], sem1
            )
        d2.wait()
        dst[p] = pltpu.async_copy(
            accs[p], out_hbm.at[pl.ds(w * NPW + sub * KK, KK)], sem3
        )
    dst[0].wait()
    dst[1].wait()

    @pl.when(w == NW - 1)
    def _tail():
        pltpu.sync_copy(
            t1.at[idx1.at[pl.ds(SUBS * KK, EXTRA)]], acc0.at[pl.ds(0, EXTRA)]
        )
        pltpu.sync_copy(
            t2.at[idx2.at[pl.ds(SUBS * KK, EXTRA)]],
            acc0.at[pl.ds(0, EXTRA)],
            add=True,
        )
        pltpu.sync_copy(
            acc0.at[pl.ds(0, EXTRA)],
            out_hbm.at[pl.ds((NW - 1) * NPW + SUBS * KK, EXTRA)],
        )


def kernel(edge_index, num_nodes, enc1, enc2):
    nn = jnp.full((L,), num_nodes, jnp.int32)
    hist = _hist_kernel(edge_index, nn)
    return _embed_kernel(hist, enc1, enc2)
